# baseline (speedup 1.0000x reference)
import jax
import jax.numpy as jnp
from jax.experimental import pallas as pl
from jax.experimental.pallas import tpu as pltpu

_CHUNK = 2304
_NT = 32
_NS = 9
_TILES = _CHUNK // 128


def _body(x_ref, wd_ref, tt_ref, st_ref, o_ref, base_ref):
    d = o_ref.shape[2]
    bi = pl.program_id(1)

    @pl.when(bi == 0)
    def _build_base():
        time_part = jnp.broadcast_to(
            tt_ref[...][:, None, :], (_NT, _CHUNK // _NT, d)).reshape(_CHUNK, d)
        space_part = jnp.broadcast_to(
            st_ref[0][:, None, :], (_NS, _CHUNK // _NS, d)).reshape(_CHUNK, d)
        base_ref[...] = time_part + space_part

    xt = x_ref[0, 0]
    wrow = wd_ref[0][None, :]
    drow = wd_ref[1][None, :]
    for i in range(_TILES):
        xcol = xt[:, i:i + 1]
        y = jnp.dot(xcol, wrow, preferred_element_type=jnp.float32)
        upd = jnp.where(jnp.isnan(y), drow, y)
        sl = pl.ds(i * 128, 128)
        o_ref[0, sl, :] = base_ref[sl, :] + upd


def kernel(x, W, b, time_table, space_table, nan_table):
    bsize, T, J, D = x.shape
    n = T * J * D
    nchunks = n // _CHUNK
    d_model = W.shape[0]
    xt4 = jnp.swapaxes(x.reshape(bsize, nchunks, _TILES, 128), -1, -2)
    st3 = space_table.reshape(nchunks, _NS, d_model)
    tt2 = time_table + b[None, :] + nan_table[0][None, :]
    wd = jnp.stack([W[:, 0], nan_table[1] - nan_table[0]], axis=0)
    grid = (nchunks, bsize)
    return pl.pallas_call(
        _body,
        grid=grid,
        in_specs=[
            pl.BlockSpec((1, 1, 128, _TILES), lambda ci, bi: (bi, ci, 0, 0)),
            pl.BlockSpec((2, d_model), lambda ci, bi: (0, 0)),
            pl.BlockSpec((_NT, d_model), lambda ci, bi: (ci, 0)),
            pl.BlockSpec((1, _NS, d_model), lambda ci, bi: (ci, 0, 0)),
        ],
        out_specs=pl.BlockSpec((1, _CHUNK, d_model), lambda ci, bi: (bi, ci, 0)),
        out_shape=jax.ShapeDtypeStruct((bsize, n, d_model), jnp.float32),
        scratch_shapes=[pltpu.VMEM((_CHUNK, d_model), jnp.float32)],
    )(xt4, wd, tt2, st3)


# speedup vs baseline: 1.2613x; 1.2613x over previous
import jax
import jax.numpy as jnp
from jax.experimental import pallas as pl
from jax.experimental.pallas import tpu as pltpu

_CHUNK = 4608
_NT = 64
_NS = 18
_TILES = _CHUNK // 128


def _body(x_ref, wd_ref, tt_ref, st_ref, o_ref, base_ref):
    d = o_ref.shape[2]
    bi = pl.program_id(1)

    @pl.when(bi == 0)
    def _build_base():
        time_part = jnp.broadcast_to(
            tt_ref[...][:, None, :], (_NT, _CHUNK // _NT, d)).reshape(_CHUNK, d)
        space_part = jnp.broadcast_to(
            st_ref[0][:, None, :], (_NS, _CHUNK // _NS, d)).reshape(_CHUNK, d)
        base_ref[...] = time_part + space_part

    xt = x_ref[0, 0]
    wrow = wd_ref[0][None, :]
    drow = wd_ref[1][None, :]
    for i in range(_TILES):
        xcol = xt[:, i:i + 1]
        y = xcol * wrow
        upd = jnp.where(jnp.isnan(y), drow, y)
        sl = pl.ds(i * 128, 128)
        o_ref[0, sl, :] = base_ref[sl, :] + upd


def kernel(x, W, b, time_table, space_table, nan_table):
    bsize, T, J, D = x.shape
    n = T * J * D
    nchunks = n // _CHUNK
    d_model = W.shape[0]
    xt4 = jnp.swapaxes(x.reshape(bsize, nchunks, _TILES, 128), -1, -2)
    st3 = space_table.reshape(nchunks, _NS, d_model)
    tt2 = time_table + b[None, :] + nan_table[0][None, :]
    wd = jnp.stack([W[:, 0], nan_table[1] - nan_table[0]], axis=0)
    grid = (nchunks, bsize)
    return pl.pallas_call(
        _body,
        grid=grid,
        in_specs=[
            pl.BlockSpec((1, 1, 128, _TILES), lambda ci, bi: (bi, ci, 0, 0)),
            pl.BlockSpec((2, d_model), lambda ci, bi: (0, 0)),
            pl.BlockSpec((_NT, d_model), lambda ci, bi: (ci, 0)),
            pl.BlockSpec((1, _NS, d_model), lambda ci, bi: (ci, 0, 0)),
        ],
        out_specs=pl.BlockSpec((1, _CHUNK, d_model), lambda ci, bi: (bi, ci, 0)),
        out_shape=jax.ShapeDtypeStruct((bsize, n, d_model), jnp.float32),
        scratch_shapes=[pltpu.VMEM((_CHUNK, d_model), jnp.float32)],
    )(xt4, wd, tt2, st3)


# speedup vs baseline: 1.4112x; 1.1189x over previous
import jax
import jax.numpy as jnp
from jax.experimental import pallas as pl
from jax.experimental.pallas import tpu as pltpu

_CHUNK = 9216
_NT = 128
_NS = 36
_TILES = _CHUNK // 128


def _body(x_ref, wd_ref, tt_ref, st_ref, o_ref, base_ref):
    d = o_ref.shape[2]
    bi = pl.program_id(1)

    @pl.when(bi == 0)
    def _build_base():
        time_part = jnp.broadcast_to(
            tt_ref[...][:, None, :], (_NT, _CHUNK // _NT, d)).reshape(_CHUNK, d)
        space_part = jnp.broadcast_to(
            st_ref[0][:, None, :], (_NS, _CHUNK // _NS, d)).reshape(_CHUNK, d)
        base_ref[...] = time_part + space_part

    xt = x_ref[0, 0]
    wrow = wd_ref[0][None, :]
    drow = wd_ref[1][None, :]
    for i in range(_TILES):
        xcol = xt[:, i:i + 1]
        y = xcol * wrow
        upd = jnp.where(jnp.isnan(y), drow, y)
        sl = pl.ds(i * 128, 128)
        o_ref[0, sl, :] = base_ref[sl, :] + upd


def kernel(x, W, b, time_table, space_table, nan_table):
    bsize, T, J, D = x.shape
    n = T * J * D
    nchunks = n // _CHUNK
    d_model = W.shape[0]
    xt4 = jnp.swapaxes(x.reshape(bsize, nchunks, _TILES, 128), -1, -2)
    st3 = space_table.reshape(nchunks, _NS, d_model)
    tt2 = time_table + b[None, :] + nan_table[0][None, :]
    wd = jnp.stack([W[:, 0], nan_table[1] - nan_table[0]], axis=0)
    grid = (nchunks, bsize)
    return pl.pallas_call(
        _body,
        grid=grid,
        in_specs=[
            pl.BlockSpec((1, 1, 128, _TILES), lambda ci, bi: (bi, ci, 0, 0)),
            pl.BlockSpec((2, d_model), lambda ci, bi: (0, 0)),
            pl.BlockSpec((_NT, d_model), lambda ci, bi: (ci, 0)),
            pl.BlockSpec((1, _NS, d_model), lambda ci, bi: (ci, 0, 0)),
        ],
        out_specs=pl.BlockSpec((1, _CHUNK, d_model), lambda ci, bi: (bi, ci, 0)),
        out_shape=jax.ShapeDtypeStruct((bsize, n, d_model), jnp.float32),
        scratch_shapes=[pltpu.VMEM((_CHUNK, d_model), jnp.float32)],
    )(xt4, wd, tt2, st3)


# speedup vs baseline: 1.4384x; 1.0192x over previous
import jax
import jax.numpy as jnp
from jax.experimental import pallas as pl
from jax.experimental.pallas import tpu as pltpu

_CHUNK = 18432
_NT = 256
_NS = 72
_TILES = _CHUNK // 128


def _body(x_ref, wd_ref, tt_ref, st_ref, o_ref, base_ref):
    d = o_ref.shape[2]
    bi = pl.program_id(1)

    @pl.when(bi == 0)
    def _build_base():
        time_part = jnp.broadcast_to(
            tt_ref[...][:, None, :], (_NT, _CHUNK // _NT, d)).reshape(_CHUNK, d)
        space_part = jnp.broadcast_to(
            st_ref[0][:, None, :], (_NS, _CHUNK // _NS, d)).reshape(_CHUNK, d)
        base_ref[...] = time_part + space_part

    xt = x_ref[0, 0]
    wrow = wd_ref[0][None, :]
    drow = wd_ref[1][None, :]
    for i in range(_TILES):
        xcol = xt[:, i:i + 1]
        y = xcol * wrow
        upd = jnp.where(jnp.isnan(y), drow, y)
        sl = pl.ds(i * 128, 128)
        o_ref[0, sl, :] = base_ref[sl, :] + upd


def kernel(x, W, b, time_table, space_table, nan_table):
    bsize, T, J, D = x.shape
    n = T * J * D
    nchunks = n // _CHUNK
    d_model = W.shape[0]
    xt4 = jnp.swapaxes(x.reshape(bsize, nchunks, _TILES, 128), -1, -2)
    st3 = space_table.reshape(nchunks, _NS, d_model)
    tt2 = time_table + b[None, :] + nan_table[0][None, :]
    wd = jnp.stack([W[:, 0], nan_table[1] - nan_table[0]], axis=0)
    grid = (nchunks, bsize)
    return pl.pallas_call(
        _body,
        grid=grid,
        in_specs=[
            pl.BlockSpec((1, 1, 128, _TILES), lambda ci, bi: (bi, ci, 0, 0)),
            pl.BlockSpec((2, d_model), lambda ci, bi: (0, 0)),
            pl.BlockSpec((_NT, d_model), lambda ci, bi: (ci, 0)),
            pl.BlockSpec((1, _NS, d_model), lambda ci, bi: (ci, 0, 0)),
        ],
        out_specs=pl.BlockSpec((1, _CHUNK, d_model), lambda ci, bi: (bi, ci, 0)),
        out_shape=jax.ShapeDtypeStruct((bsize, n, d_model), jnp.float32),
        scratch_shapes=[pltpu.VMEM((_CHUNK, d_model), jnp.float32)],
    )(xt4, wd, tt2, st3)
